# baseline (speedup 1.0000x reference)
import jax
import jax.numpy as jnp
from jax.experimental import pallas as pl
from jax.experimental.pallas import tpu as pltpu

_D = 1024
_S = 2048
_BS = 256
_NB = _S // _BS


def _body(s_ref, emb_ref, spe_ref, pos_ref, out_ref):
    i = pl.program_id(0)

    @pl.when(i < _NB)
    def _copy():
        out_ref[...] = emb_ref[...]

    @pl.when(i == _NB)
    def _tail():
        row = spe_ref[0, 0, :] + pos_ref[0, 0, :]
        out_ref[:, 0, :] = jnp.broadcast_to(row[None, :], (2, _D))


def kernel(bos_token, embeds, idx, speech_emb, pos_emb):
    s = jnp.concatenate([bos_token.reshape(-1), idx.reshape(-1)]).astype(jnp.int32)
    spe3 = speech_emb.reshape(speech_emb.shape[0], 1, _D)
    pos3 = pos_emb.reshape(pos_emb.shape[0], 1, _D)
    grid_spec = pltpu.PrefetchScalarGridSpec(
        num_scalar_prefetch=1,
        grid=(_NB + 1,),
        in_specs=[
            pl.BlockSpec((2, _BS, _D), lambda i, s: (0, jnp.minimum(i, _NB - 1), 0)),
            pl.BlockSpec((1, 1, _D), lambda i, s: (s[0], 0, 0)),
            pl.BlockSpec((1, 1, _D), lambda i, s: (s[1], 0, 0)),
        ],
        out_specs=pl.BlockSpec((2, _BS, _D), lambda i, s: (0, i, 0)),
    )
    return pl.pallas_call(
        _body,
        grid_spec=grid_spec,
        out_shape=jax.ShapeDtypeStruct((2, _S + 1, _D), jnp.float32),
    )(s, embeds, spe3, pos3)
